# baseline (speedup 1.0000x reference)
import jax
import jax.numpy as jnp
from jax import lax
from jax.experimental import pallas as pl
from jax.experimental.pallas import tpu as pltpu
from jax.experimental.pallas import tpu_sc as plsc

_NC = 2
_NS = 16
_NW = _NC * _NS
_CHUNK = 128
_NB = 4
_K = 2


def _gather_body(table_hbm, idxt_hbm, v_hbm, idx_v, rows_v, t_v, *sems):
    n = 16384
    d = rows_v.shape[2]
    b_per_w = idxt_hbm.shape[0] // _NW
    n_chunks = b_per_w // _CHUNK
    tiles_n = n // _CHUNK
    rows_per_j = d * tiles_n
    wid = lax.axis_index("s") * _NC + lax.axis_index("c")
    base = wid * b_per_w
    pltpu.sync_copy(idxt_hbm.at[pl.ds(base, b_per_w)], idx_v)

    lanes = jnp.arange(16, dtype=jnp.int32)
    row_iv = [lanes + il0 * 16 for il0 in range(8)]

    def idx_slice(c):
        return idx_v.at[pl.ds(c * _CHUNK, _CHUNK)]

    def fire_gather_slot(c, g):
        pltpu.async_copy(table_hbm.at[idx_slice(c)], rows_v.at[g], sems[g])

    def wait_gather(g):
        pltpu.make_async_copy(
            table_hbm.at[idx_v.at[pl.ds(0, _CHUNK)]], rows_v.at[g],
            sems[g]).wait()

    def transpose(s):
        @plsc.parallel_loop(0, d, step=1, unroll=16)
        def _(c_out):
            col = jnp.zeros((16,), jnp.int32) + c_out
            for il0 in range(8):
                vals = plsc.load_gather(rows_v.at[s], [row_iv[il0], col])
                t_v[s, c_out, pl.ds(il0 * 16, 16)] = vals

    def fire_stores(c, s):
        q = base + c * _CHUNK
        r0 = (q // n) * rows_per_j + ((q % n) // _CHUNK) * 8
        for tc in range(d // 8):
            pltpu.async_copy(
                t_v.at[s, pl.ds(tc * 8, 8), :],
                v_hbm.at[pl.ds(r0 + tc * tiles_n * 8, 8)],
                sems[_NB + s])

    def wait_stores(s):
        pltpu.make_async_copy(
            t_v.at[s], v_hbm.at[pl.ds(0, d)], sems[_NB + s]).wait()

    def visit(c, s, do_wait_stores, do_fire):
        wait_gather(s)
        if do_fire:
            fire_gather_slot(c + _K, (s + _K) % _NB)
        if do_wait_stores:
            wait_stores(s)
        transpose(s)
        fire_stores(c, s)

    for c in range(_K):
        fire_gather_slot(c, c % _NB)
    for c in range(_NB):
        visit(c, c % _NB, False, True)

    steady_end = ((n_chunks - _K) // _NB) * _NB

    def group(i, carry):
        c0 = _NB + i * _NB
        for u in range(_NB):
            visit(c0 + u, u, True, True)
        return carry

    lax.fori_loop(0, (steady_end - _NB) // _NB, group, 0)

    for c in range(steady_end, n_chunks):
        visit(c, c % _NB, True, c + _K < n_chunks)

    for s in range(_NB):
        wait_stores(s)


def _idxt_body(idx_hbm, out_hbm, buf_v, tbuf_v, sem):
    n, k = idx_hbm.shape
    nblk = n // 128 // _NW
    strip = nblk * 128
    wid = lax.axis_index("s") * _NC + lax.axis_index("c")
    i0 = wid * strip
    lanes = jnp.arange(16, dtype=jnp.int32)
    row_iv = [lanes + il0 * 16 for il0 in range(8)]

    for t in range(nblk):
        pltpu.sync_copy(idx_hbm.at[pl.ds(i0 + t * 128, 128)], buf_v)

        @plsc.parallel_loop(0, k, step=1, unroll=5)
        def _(j):
            col = jnp.zeros((16,), jnp.int32) + j
            for il0 in range(8):
                vals = plsc.load_gather(buf_v, [row_iv[il0], col])
                tbuf_v[j, pl.ds(t * 128 + il0 * 16, 16)] = vals

    def fire(j, carry):
        pltpu.async_copy(
            tbuf_v.at[j], out_hbm.at[pl.ds(j * n + i0, strip)], sem)
        return carry

    lax.fori_loop(0, k, fire, 0)

    def drain(j, carry):
        pltpu.make_async_copy(
            tbuf_v.at[0], out_hbm.at[pl.ds(0, strip)], sem).wait()
        return carry

    lax.fori_loop(0, k, drain, 0)


def kernel(data, indices):
    n, k = indices.shape
    d = data.shape[1]
    b = n * k
    mesh0 = plsc.VectorSubcoreMesh(core_axis_name="c", subcore_axis_name="s")
    idxt = pl.kernel(
        _idxt_body,
        out_type=jax.ShapeDtypeStruct((b,), jnp.int32),
        mesh=mesh0,
        scratch_types=[
            pltpu.VMEM((128, k), jnp.int32),
            pltpu.VMEM((k, n // _NW), jnp.int32),
            pltpu.SemaphoreType.DMA,
        ],
        compiler_params=pltpu.CompilerParams(
            use_tc_tiling_on_sc=False, needs_layout_passes=False),
    )(indices)
    v_rows = b * d // 128
    mesh = plsc.VectorSubcoreMesh(core_axis_name="c", subcore_axis_name="s")
    v = pl.kernel(
        _gather_body,
        out_type=jax.ShapeDtypeStruct((v_rows, 128), jnp.float32),
        mesh=mesh,
        scratch_types=[
            pltpu.VMEM((b // _NW,), jnp.int32),
            pltpu.VMEM((_NB, _CHUNK, d), jnp.float32),
            pltpu.VMEM((_NB, d, _CHUNK), jnp.float32),
        ] + [pltpu.SemaphoreType.DMA] * (2 * _NB),
        compiler_params=pltpu.CompilerParams(
            use_tc_tiling_on_sc=False, needs_layout_passes=False),
    )(data, idxt)
    return (v.reshape(k, d // 8, n // 128, 8, 128)
            .transpose(2, 4, 0, 1, 3).reshape(n, k, d))


# speedup vs baseline: 1.0813x; 1.0813x over previous
import jax
import jax.numpy as jnp
from jax import lax
from jax.experimental import pallas as pl
from jax.experimental.pallas import tpu as pltpu
from jax.experimental.pallas import tpu_sc as plsc

_NC = 2
_NS = 16
_NW = _NC * _NS
_CHUNK = 128
_NBUF = 8
_K = 4


def _gather_body(table_hbm, idxt_hbm, out_hbm, idx_v, rows_v, *sems):
    k, n = idxt_hbm.shape
    d = rows_v.shape[2]
    b_per_w = (n * k) // _NW
    n_chunks = b_per_w // _CHUNK
    n_rows = idx_v.shape[0]
    wid = lax.axis_index("s") * _NC + lax.axis_index("c")
    base = wid * b_per_w
    j0 = jnp.minimum(base // n, k - n_rows)
    pltpu.sync_copy(idxt_hbm.at[pl.ds(j0, n_rows)], idx_v)

    def idx_slice(c):
        q = base + c * _CHUNK
        return idx_v.at[q // n - j0, pl.ds(q % n, _CHUNK)]

    def fire_gather(c, b):
        pltpu.async_copy(table_hbm.at[idx_slice(c)], rows_v.at[b], sems[b])

    def wait_gather(b):
        pltpu.make_async_copy(
            table_hbm.at[idx_v.at[0, pl.ds(0, _CHUNK)]], rows_v.at[b],
            sems[b]).wait()

    def fire_store(c, b):
        pltpu.async_copy(
            rows_v.at[b], out_hbm.at[pl.ds(base + c * _CHUNK, _CHUNK)],
            sems[b])

    def wait_store(b):
        pltpu.make_async_copy(
            rows_v.at[b], out_hbm.at[pl.ds(base, _CHUNK)], sems[b]).wait()

    for c in range(_K):
        fire_gather(c, c % _NBUF)

    for c in range(_NBUF - _K):
        wait_gather(c % _NBUF)
        fire_store(c, c % _NBUF)
        fire_gather(c + _K, (c + _K) % _NBUF)

    lo = _NBUF - _K
    hi = n_chunks - _K
    n_groups = (hi - lo) // _NBUF

    def group(i, carry):
        c0 = lo + i * _NBUF
        for j in range(_NBUF):
            c = c0 + j
            b = (lo + j) % _NBUF
            bf = (lo + j + _K) % _NBUF
            wait_gather(b)
            fire_store(c, b)
            wait_store(bf)
            fire_gather(c + _K, bf)
        return carry

    lax.fori_loop(0, n_groups, group, 0)

    for c in range(n_chunks - _K, n_chunks):
        wait_gather(c % _NBUF)
        fire_store(c, c % _NBUF)

    for b in range(_NBUF):
        wait_store(b)


def kernel(data, indices):
    n, k = indices.shape
    d = data.shape[1]
    b = n * k
    idxt = indices.T.astype(jnp.int32)
    n_rows = (b // _NW) // n + 2
    mesh = plsc.VectorSubcoreMesh(core_axis_name="c", subcore_axis_name="s")
    out = pl.kernel(
        _gather_body,
        out_type=jax.ShapeDtypeStruct((b, d), jnp.float32),
        mesh=mesh,
        scratch_types=[
            pltpu.VMEM((n_rows, n), jnp.int32),
            pltpu.VMEM((_NBUF, _CHUNK, d), jnp.float32),
        ] + [pltpu.SemaphoreType.DMA] * _NBUF,
        compiler_params=pltpu.CompilerParams(use_tc_tiling_on_sc=False),
    )(data, idxt)
    return out.reshape(k, n, d).transpose(1, 0, 2)
